# Initial kernel scaffold; baseline (speedup 1.0000x reference)
import jax
import jax.numpy as jnp
from jax.experimental import pallas as pl


def kernel(x, fake_pos, pin_feature, edge_index, batch, macro_index, W1, Wp, b1, W2, b2, W3, b3, M1, mb1, M2, mb2, M3, mb3):
    raise NotImplementedError("write your pallas kernel here")


# speedup vs baseline: 1.0144x; 1.0144x over previous
import jax
import jax.numpy as jnp
from jax.experimental import pallas as pl
from jax.experimental.pallas import tpu as pltpu

N = 10000
E = 320000
HE = 10000
F_IN = 125
NHID = 128
NCLS = 4
G = 16
NMACRO = 500
DPIN = 4
SLOPE = 0.1


def _leaky(v):
    return jnp.where(v >= 0, v, SLOPE * v)


def _mlp_kernel(z_ref, M1_ref, mb1_ref, M2_ref, mb2_ref, M3_ref, mb3_ref, o_ref):
    z = z_ref[...]
    z = _leaky(jnp.dot(z, M1_ref[...], preferred_element_type=jnp.float32) + mb1_ref[...])
    z = _leaky(jnp.dot(z, M2_ref[...], preferred_element_type=jnp.float32) + mb2_ref[...])
    o_ref[...] = jnp.dot(z, M3_ref[...], preferred_element_type=jnp.float32) + mb3_ref[...]


def kernel(x, fake_pos, pin_feature, edge_index, batch, macro_index, W1, Wp, b1, W2, b2, W3, b3, M1, mb1, M2, mb2, M3, mb3):
    row, col = edge_index[0], edge_index[1]

    ones_e = jnp.ones((E,), jnp.float32)
    Bdeg = jnp.zeros((HE,), jnp.float32).at[col].add(1.0)
    Ddeg = jnp.zeros((N,), jnp.float32).at[row].add(1.0)
    Binv = jnp.where(Bdeg > 0, 1.0 / Bdeg, 0.0)
    Dinv = jnp.where(Ddeg > 0, 1.0 / Ddeg, 0.0)
    cnt = jnp.zeros((N,), jnp.float32).at[macro_index].add(1.0)
    ismacro = jnp.minimum(cnt, 1.0)[:, None]
    onehot = (batch[None, :] == jnp.arange(G, dtype=jnp.int32)[:, None]).astype(jnp.float32)
    onehotm = onehot * cnt[None, :]
    ca = jnp.maximum(onehot.sum(axis=1), 1.0)
    cm = jnp.maximum(onehotm.sum(axis=1), 1.0)
    P = jnp.zeros((HE, DPIN), jnp.float32).at[col].add(pin_feature)
    EP = P @ Wp

    h0 = jnp.concatenate([x, fake_pos, ismacro], axis=-1)

    def conv(h, W, b, ep):
        xt = h @ W
        e = (jnp.zeros((HE, NHID), jnp.float32).at[col].add(xt[row]) + ep) * Binv[:, None]
        out = jnp.zeros((N, NHID), jnp.float32).at[row].add(e[col]) * Dinv[:, None]
        return _leaky(out + b)

    def pool(h):
        gm = (onehotm @ h) / cm[:, None]
        gs = (onehot @ h) / ca[:, None]
        return jnp.concatenate([gm, gs], axis=1)

    zero_ep = jnp.zeros((HE, NHID), jnp.float32)
    h = conv(h0, W1, b1, EP)
    z = pool(h)
    h = conv(h, W2, b2, zero_ep)
    z = z + pool(h)
    h = conv(h, W3, b3, zero_ep)
    z = z + pool(h)

    out = pl.pallas_call(
        _mlp_kernel,
        out_shape=jax.ShapeDtypeStruct((G, NCLS), jnp.float32),
    )(z, M1, mb1, M2, mb2, M3, mb3)
    return out


# speedup vs baseline: 4.3540x; 4.2921x over previous
import jax
import jax.numpy as jnp
from jax import lax
from jax.experimental import pallas as pl
from jax.experimental.pallas import tpu as pltpu
from jax.experimental.pallas import tpu_sc as plsc

N = 10000
E = 320000
HE = 10000
NHID = 128
H = 64
G = 16
NCLS = 4
SLOPE = 0.1

NC = 2
NS = 16
K = 80
NCH = E // K
CH_W = NCH // NS
CH_P = NCH // (NC * NS)
RPT = N // NS
NP16 = 10016
F32 = jnp.float32

_mesh = plsc.VectorSubcoreMesh(core_axis_name="c", subcore_axis_name="s")
_sc_params = pltpu.CompilerParams(use_tc_tiling_on_sc=False)


def _leaky(v):
    return jnp.where(v >= 0, v, SLOPE * v)


HI = jax.lax.Precision.HIGHEST


def _sc_pre_body(ridx, cidx, pin16, macro_pad,
                 degB_o, degD_o, cntM_o, P16_o,
                 accB, accD, accC, accP, rv, cv, pbuf, ones, zbuf, mbuf):
    c = lax.axis_index("c")
    s = lax.axis_index("s")
    w = c * NS + s

    pltpu.sync_copy(ridx.at[pl.ds(w * CH_P, CH_P)], rv)
    pltpu.sync_copy(cidx.at[pl.ds(w * CH_P, CH_P)], cv)

    @pl.loop(0, 626)
    def _(r):
        zbuf[r, pl.ds(0, 16)] = jnp.zeros((16,), F32)

    @pl.loop(0, K)
    def _(r):
        ones[r, pl.ds(0, 16)] = jnp.ones((16,), F32)

    for acc in (accB, accD, accC, accP):
        pltpu.sync_copy(zbuf, acc.at[pl.ds(s * 626, 626)])
    plsc.subcore_barrier()

    @pl.loop(0, CH_P)
    def _(ch):
        pltpu.sync_copy(pin16.at[pl.ds((w * CH_P + ch) * K, K)], pbuf)
        pltpu.sync_copy(ones, accD.at[rv.at[ch]], add=True)
        pltpu.sync_copy(ones, accB.at[cv.at[ch]], add=True)
        pltpu.sync_copy(pbuf, accP.at[cv.at[ch]], add=True)

    @pl.when(c == 0)
    def _():
        pltpu.sync_copy(macro_pad.at[s], mbuf)
        pltpu.sync_copy(ones.at[pl.ds(0, 32)], accC.at[mbuf], add=True)

    plsc.subcore_barrier()
    sl = pl.ds(s * RPT, RPT)
    pltpu.sync_copy(accB.at[sl], degB_o.at[c, sl])
    pltpu.sync_copy(accD.at[sl], degD_o.at[c, sl])
    pltpu.sync_copy(accC.at[sl], cntM_o.at[c, sl])
    pltpu.sync_copy(accP.at[sl], P16_o.at[c, sl])


def _sc_precompute(ridx, cidx, pin16, macro_pad):
    out = jax.ShapeDtypeStruct((NC, HE, 16), F32)
    return pl.kernel(
        _sc_pre_body,
        out_type=[out, out, out, out],
        mesh=_mesh,
        compiler_params=_sc_params,
        scratch_types=[
            pltpu.VMEM_SHARED((NP16, 16), F32),
            pltpu.VMEM_SHARED((NP16, 16), F32),
            pltpu.VMEM_SHARED((NP16, 16), F32),
            pltpu.VMEM_SHARED((NP16, 16), F32),
            pltpu.VMEM((CH_P, K), jnp.int32),
            pltpu.VMEM((CH_P, K), jnp.int32),
            pltpu.VMEM((K, 16), F32),
            pltpu.VMEM((K, 16), F32),
            pltpu.VMEM((626, 16), F32),
            pltpu.VMEM((32,), jnp.int32),
        ],
    )(ridx, cidx, pin16, macro_pad)


IB = 25


def _sc_layer_body(x0, x1, ridx, cidx, einit, zer, binv_pad,
                   o2,
                   e_sh, o_sh, rv, cv, rows, esl, binv_v):
    c = lax.axis_index("c")
    s = lax.axis_index("s")

    pltpu.sync_copy(binv_pad.at[s], binv_v)

    sl = pl.ds(s * RPT, RPT)
    pltpu.sync_copy(einit.at[c, sl], e_sh.at[sl])
    pltpu.sync_copy(zer.at[sl], o_sh.at[sl])
    plsc.subcore_barrier()

    @pl.loop(0, CH_W // IB)
    def _(blk):
        bs = s * CH_W + blk * IB
        pltpu.sync_copy(ridx.at[pl.ds(bs, IB)], rv)
        pltpu.sync_copy(cidx.at[pl.ds(bs, IB)], cv)

        @pl.when(c == 0)
        def _():
            @pl.loop(0, IB)
            def _(ch):
                pltpu.sync_copy(x0.at[rv.at[ch]], rows)
                pltpu.sync_copy(rows, e_sh.at[cv.at[ch]], add=True)

        @pl.when(c == 1)
        def _():
            @pl.loop(0, IB)
            def _(ch):
                pltpu.sync_copy(x1.at[rv.at[ch]], rows)
                pltpu.sync_copy(rows, e_sh.at[cv.at[ch]], add=True)

    plsc.subcore_barrier()

    pltpu.sync_copy(e_sh.at[sl], esl)

    @pl.loop(0, RPT // 16)
    def _(t):
        vv = binv_v[pl.ds(t * 16, 16)]
        for k in range(16):
            sc = vv[k]
            r = t * 16 + k
            for j in range(H // 16):
                esl[r, pl.ds(j * 16, 16)] = esl[r, pl.ds(j * 16, 16)] * sc

    vv = binv_v[pl.ds(RPT - 1, 16)]
    sc = vv[0]
    for j in range(H // 16):
        esl[RPT - 1, pl.ds(j * 16, 16)] = esl[RPT - 1, pl.ds(j * 16, 16)] * sc

    pltpu.sync_copy(esl, e_sh.at[sl])
    plsc.subcore_barrier()

    @pl.loop(0, CH_W // IB)
    def _(blk):
        bs = s * CH_W + blk * IB
        pltpu.sync_copy(ridx.at[pl.ds(bs, IB)], rv)
        pltpu.sync_copy(cidx.at[pl.ds(bs, IB)], cv)

        @pl.loop(0, IB)
        def _(ch):
            pltpu.sync_copy(e_sh.at[cv.at[ch]], rows)
            pltpu.sync_copy(rows, o_sh.at[rv.at[ch]], add=True)

    plsc.subcore_barrier()
    pltpu.sync_copy(o_sh.at[sl], o2.at[c, sl])


def _sc_layer(x0, x1, ridx, cidx, einit, zer, binv_pad):
    return pl.kernel(
        _sc_layer_body,
        out_type=jax.ShapeDtypeStruct((NC, N, H), F32),
        mesh=_mesh,
        compiler_params=_sc_params,
        scratch_types=[
            pltpu.VMEM_SHARED((HE, H), F32),
            pltpu.VMEM_SHARED((N, H), F32),
            pltpu.VMEM((IB, K), jnp.int32),
            pltpu.VMEM((IB, K), jnp.int32),
            pltpu.VMEM((K, H), F32),
            pltpu.VMEM((RPT, H), F32),
            pltpu.VMEM((640,), F32),
        ],
    )(x0, x1, ridx, cidx, einit, zer, binv_pad)


BN = 2000
NB = N // BN


def _tc_prep_body(xpad, fp, degB, degD, cntM, P16, batch, W1x, wfp, wm, Wp,
                  xts, EP, binv, dinv, cnt1, caa, cma):
    i = pl.program_id(0)
    cnt = cntM[0, :, 0:1] + cntM[1, :, 0:1]
    ism = jnp.minimum(cnt, 1.0)
    xt = (jnp.dot(xpad[...], W1x[...], precision=HI, preferred_element_type=F32)
          + fp[:, 0:1] * wfp[0:1, :] + fp[:, 1:2] * wfp[1:2, :] + ism * wm[...])
    xts[0] = xt[:, :H]
    xts[1] = xt[:, H:]
    Bdeg = degB[0, :, 0:1] + degB[1, :, 0:1]
    binv[...] = jnp.where(Bdeg > 0, 1.0 / Bdeg, 0.0)
    Ddeg = degD[0, :, 0:1] + degD[1, :, 0:1]
    dinv[...] = jnp.where(Ddeg > 0, 1.0 / Ddeg, 0.0)
    cnt1[...] = cnt
    P = P16[0, :, 0:4] + P16[1, :, 0:4]
    ep = jnp.dot(P, Wp[...], precision=HI, preferred_element_type=F32)
    EP[0] = ep[:, :H]
    EP[1] = ep[:, H:]
    oh = (lax.broadcasted_iota(jnp.int32, (G, BN), 0) == batch[0]).astype(F32)

    @pl.when(i == 0)
    def _():
        caa[...] = jnp.zeros((G, 1), F32)
        cma[...] = jnp.zeros((G, 1), F32)

    caa[...] += jnp.sum(oh, axis=1, keepdims=True)
    cma[...] += jnp.dot(oh, cnt, precision=HI, preferred_element_type=F32)

    @pl.when(i == NB - 1)
    def _():
        caa[...] = jnp.maximum(caa[...], 1.0)
        cma[...] = jnp.maximum(cma[...], 1.0)


def _tc_prep(xpad, fp, degB, degD, cntM, P16, batch1, W1x, wfp, wm, Wp):
    bs3 = pl.BlockSpec((NC, BN, 16), lambda i: (0, i, 0))
    return pl.pallas_call(
        _tc_prep_body,
        grid=(NB,),
        in_specs=[
            pl.BlockSpec((BN, 128), lambda i: (i, 0)),
            pl.BlockSpec((BN, 2), lambda i: (i, 0)),
            bs3, bs3, bs3, bs3,
            pl.BlockSpec((1, 1, BN), lambda i: (i, 0, 0)),
            pl.BlockSpec((128, NHID), lambda i: (0, 0)),
            pl.BlockSpec((2, NHID), lambda i: (0, 0)),
            pl.BlockSpec((1, NHID), lambda i: (0, 0)),
            pl.BlockSpec((4, NHID), lambda i: (0, 0)),
        ],
        out_specs=[
            pl.BlockSpec((NC, BN, H), lambda i: (0, i, 0)),
            pl.BlockSpec((NC, BN, H), lambda i: (0, i, 0)),
            pl.BlockSpec((BN, 1), lambda i: (i, 0)),
            pl.BlockSpec((BN, 1), lambda i: (i, 0)),
            pl.BlockSpec((BN, 1), lambda i: (i, 0)),
            pl.BlockSpec((G, 1), lambda i: (0, 0)),
            pl.BlockSpec((G, 1), lambda i: (0, 0)),
        ],
        out_shape=[
            jax.ShapeDtypeStruct((NC, N, H), F32),
            jax.ShapeDtypeStruct((NC, HE, H), F32),
            jax.ShapeDtypeStruct((N, 1), F32),
            jax.ShapeDtypeStruct((N, 1), F32),
            jax.ShapeDtypeStruct((N, 1), F32),
            jax.ShapeDtypeStruct((G, 1), F32),
            jax.ShapeDtypeStruct((G, 1), F32),
        ],
    )(xpad, fp, degB, degD, cntM, P16, batch1, W1x, wfp, wm, Wp)


def _pool_accum(i, oh, cnt, h0b, h1b, praw):
    @pl.when(i == 0)
    def _():
        praw[...] = jnp.zeros((4, G, H), F32)

    ch0 = cnt * h0b
    ch1 = cnt * h1b
    praw[0] += jnp.dot(oh, ch0, precision=HI, preferred_element_type=F32)
    praw[1] += jnp.dot(oh, ch1, precision=HI, preferred_element_type=F32)
    praw[2] += jnp.dot(oh, h0b, precision=HI, preferred_element_type=F32)
    praw[3] += jnp.dot(oh, h1b, precision=HI, preferred_element_type=F32)


def _tc_mid_body(o2, dinv, cnt1, batch, b, W, xts, praw):
    i = pl.program_id(0)
    dv = dinv[...]
    h0b = _leaky(o2[0] * dv + b[:, 0:H])
    h1b = _leaky(o2[1] * dv + b[:, H:])
    xts[0] = (jnp.dot(h0b, W[0:H, 0:H], precision=HI, preferred_element_type=F32)
              + jnp.dot(h1b, W[H:, 0:H], precision=HI, preferred_element_type=F32))
    xts[1] = (jnp.dot(h0b, W[0:H, H:], precision=HI, preferred_element_type=F32)
              + jnp.dot(h1b, W[H:, H:], precision=HI, preferred_element_type=F32))
    oh = (lax.broadcasted_iota(jnp.int32, (G, BN), 0) == batch[0]).astype(F32)
    _pool_accum(i, oh, cnt1[...], h0b, h1b, praw)


def _tc_last_body(o2, dinv, cnt1, batch, b, xts, praw):
    i = pl.program_id(0)
    dv = dinv[...]
    h0b = _leaky(o2[0] * dv + b[:, 0:H])
    h1b = _leaky(o2[1] * dv + b[:, H:])
    xts[0] = h0b
    xts[1] = h1b
    oh = (lax.broadcasted_iota(jnp.int32, (G, BN), 0) == batch[0]).astype(F32)
    _pool_accum(i, oh, cnt1[...], h0b, h1b, praw)


def _tc_layer_end(body, o2, dinv, cnt1, batch1, b, *Ws):
    in_specs = [
        pl.BlockSpec((NC, BN, H), lambda i: (0, i, 0)),
        pl.BlockSpec((BN, 1), lambda i: (i, 0)),
        pl.BlockSpec((BN, 1), lambda i: (i, 0)),
        pl.BlockSpec((1, 1, BN), lambda i: (i, 0, 0)),
        pl.BlockSpec((1, NHID), lambda i: (0, 0)),
    ] + [pl.BlockSpec((NHID, NHID), lambda i: (0, 0)) for _ in Ws]
    return pl.pallas_call(
        body,
        grid=(NB,),
        in_specs=in_specs,
        out_specs=[
            pl.BlockSpec((NC, BN, H), lambda i: (0, i, 0)),
            pl.BlockSpec((4, G, H), lambda i: (0, 0, 0)),
        ],
        out_shape=[
            jax.ShapeDtypeStruct((NC, N, H), F32),
            jax.ShapeDtypeStruct((4, G, H), F32),
        ],
    )(o2, dinv, cnt1, batch1, b, *Ws)


def _tc_final_body(p1, p2, p3, caa, cma, M1, mb1, M2, mb2, M3, mb3, out):
    t = jnp.zeros((G, NHID), F32)
    for q in range(4):
        zq = p1[q] + p2[q] + p3[q]
        zq = zq / (cma[...] if q < 2 else caa[...])
        t += jnp.dot(zq, M1[q * H:(q + 1) * H, :], precision=HI,
                     preferred_element_type=F32)
    t = _leaky(t + mb1[...])
    t = _leaky(jnp.dot(t, M2[...], precision=HI, preferred_element_type=F32)
               + mb2[...])
    out[...] = (jnp.dot(t, M3[...], precision=HI, preferred_element_type=F32)
                + mb3[...])


def _tc_final(p1, p2, p3, caa, cma, M1, mb1, M2, mb2, M3, mb3):
    return pl.pallas_call(
        _tc_final_body,
        out_shape=jax.ShapeDtypeStruct((G, NCLS), F32),
    )(p1, p2, p3, caa, cma, M1, mb1, M2, mb2, M3, mb3)


def kernel(x, fake_pos, pin_feature, edge_index, batch, macro_index,
           W1, Wp, b1, W2, b2, W3, b3, M1, mb1, M2, mb2, M3, mb3):
    row, col = edge_index[0], edge_index[1]

    xpad = jnp.pad(x, ((0, 0), (0, 3)))
    W1x = jnp.concatenate([W1[:125], jnp.zeros((3, NHID), F32)], axis=0)
    wfp = W1[125:127]
    wm = W1[127:128]
    batch1 = batch.reshape(NB, 1, BN)
    ridx = row.reshape(NCH, K)
    cidx = col.reshape(NCH, K)
    pin16 = jnp.pad(pin_feature, ((0, 0), (0, 12)))
    macro_pad = jnp.concatenate(
        [macro_index, jnp.full((12,), N, jnp.int32)]).reshape(NS, 32)
    zer = jnp.zeros((N, H), F32)
    zeros_e = jnp.zeros((NC, HE, H), F32)
    b1r, b2r, b3r = b1.reshape(1, NHID), b2.reshape(1, NHID), b3.reshape(1, NHID)
    mb1r, mb2r, mb3r = mb1.reshape(1, NHID), mb2.reshape(1, H), mb3.reshape(1, NCLS)

    degB, degD, cntM, P16 = _sc_precompute(ridx, cidx, pin16, macro_pad)
    xts1, EP, binv1, dinv1, cnt1, caa, cma = _tc_prep(
        xpad, fake_pos, degB, degD, cntM, P16, batch1, W1x, wfp, wm, Wp)
    binv_pad = jnp.pad(binv1.reshape(NS, RPT), ((0, 0), (0, 640 - RPT)))

    o2 = _sc_layer(xts1[0], xts1[1], ridx, cidx, EP, zer, binv_pad)
    xts2, praw1 = _tc_layer_end(_tc_mid_body, o2, dinv1, cnt1, batch1, b1r, W2)
    o2 = _sc_layer(xts2[0], xts2[1], ridx, cidx, zeros_e, zer, binv_pad)
    xts3, praw2 = _tc_layer_end(_tc_mid_body, o2, dinv1, cnt1, batch1, b2r, W3)
    o2 = _sc_layer(xts3[0], xts3[1], ridx, cidx, zeros_e, zer, binv_pad)
    _, praw3 = _tc_layer_end(_tc_last_body, o2, dinv1, cnt1, batch1, b3r)

    return _tc_final(praw1, praw2, praw3, caa, cma, M1, mb1r, M2, mb2r, M3, mb3r)


# speedup vs baseline: 6.1768x; 1.4186x over previous
import jax
import jax.numpy as jnp
from jax import lax
from jax.experimental import pallas as pl
from jax.experimental.pallas import tpu as pltpu
from jax.experimental.pallas import tpu_sc as plsc

N = 10000
E = 320000
HE = 10000
NHID = 128
H = 64
G = 16
NCLS = 4
SLOPE = 0.1

NC = 2
NS = 16
K = 80
NCH = E // K
CH_W = NCH // NS
CH_P = NCH // (NC * NS)
RPT = N // NS
NP16 = 10016
F32 = jnp.float32

_mesh = plsc.VectorSubcoreMesh(core_axis_name="c", subcore_axis_name="s")
_sc_params = pltpu.CompilerParams(use_tc_tiling_on_sc=False)


def _leaky(v):
    return jnp.where(v >= 0, v, SLOPE * v)


HI = jax.lax.Precision.HIGHEST


def _sc_pre_body(ridx, cidx, pin16, macro_pad,
                 degB_o, degD_o, cntM_o, P16_o,
                 accB, accD, accC, accP, rv, cv, pbuf, ones, zbuf, mbuf):
    c = lax.axis_index("c")
    s = lax.axis_index("s")
    w = c * NS + s

    pltpu.sync_copy(ridx.at[pl.ds(w * CH_P, CH_P)], rv)
    pltpu.sync_copy(cidx.at[pl.ds(w * CH_P, CH_P)], cv)

    @pl.loop(0, 626)
    def _(r):
        zbuf[r, pl.ds(0, 16)] = jnp.zeros((16,), F32)

    @pl.loop(0, K)
    def _(r):
        ones[r, pl.ds(0, 16)] = jnp.ones((16,), F32)

    for acc in (accB, accD, accC, accP):
        pltpu.sync_copy(zbuf, acc.at[pl.ds(s * 626, 626)])
    plsc.subcore_barrier()

    @pl.loop(0, CH_P)
    def _(ch):
        pltpu.sync_copy(pin16.at[pl.ds((w * CH_P + ch) * K, K)], pbuf)
        pltpu.sync_copy(ones, accD.at[rv.at[ch]], add=True)
        pltpu.sync_copy(ones, accB.at[cv.at[ch]], add=True)
        pltpu.sync_copy(pbuf, accP.at[cv.at[ch]], add=True)

    @pl.when(c == 0)
    def _():
        pltpu.sync_copy(macro_pad.at[s], mbuf)
        pltpu.sync_copy(ones.at[pl.ds(0, 32)], accC.at[mbuf], add=True)

    plsc.subcore_barrier()
    sl = pl.ds(s * RPT, RPT)
    pltpu.sync_copy(accB.at[sl], degB_o.at[c, sl])
    pltpu.sync_copy(accD.at[sl], degD_o.at[c, sl])
    pltpu.sync_copy(accC.at[sl], cntM_o.at[c, sl])
    pltpu.sync_copy(accP.at[sl], P16_o.at[c, sl])


def _sc_precompute(ridx, cidx, pin16, macro_pad):
    out = jax.ShapeDtypeStruct((NC, HE, 16), F32)
    return pl.kernel(
        _sc_pre_body,
        out_type=[out, out, out, out],
        mesh=_mesh,
        compiler_params=_sc_params,
        scratch_types=[
            pltpu.VMEM_SHARED((NP16, 16), F32),
            pltpu.VMEM_SHARED((NP16, 16), F32),
            pltpu.VMEM_SHARED((NP16, 16), F32),
            pltpu.VMEM_SHARED((NP16, 16), F32),
            pltpu.VMEM((CH_P, K), jnp.int32),
            pltpu.VMEM((CH_P, K), jnp.int32),
            pltpu.VMEM((K, 16), F32),
            pltpu.VMEM((K, 16), F32),
            pltpu.VMEM((626, 16), F32),
            pltpu.VMEM((32,), jnp.int32),
        ],
    )(ridx, cidx, pin16, macro_pad)


IB = 25
NBUF = 5
ESL = 125


def _ring_phase(src, dst, gidx_hbm, sidx_hbm, gv, sv, rows, gsem, base):
    def body(blk):
        pltpu.sync_copy(gidx_hbm.at[pl.ds(base + blk * IB, IB)], gv)
        pltpu.sync_copy(sidx_hbm.at[pl.ds(base + blk * IB, IB)], sv)
        descs = [None] * IB
        for j in range(NBUF - 1):
            descs[j] = pltpu.async_copy(
                src.at[gv.at[j]], rows.at[j % NBUF], gsem.at[j % NBUF])
        for j in range(IB):
            descs[j].wait()
            nxt = j + NBUF - 1
            if nxt < IB:
                descs[nxt] = pltpu.async_copy(
                    src.at[gv.at[nxt]], rows.at[nxt % NBUF],
                    gsem.at[nxt % NBUF])
            pltpu.sync_copy(rows.at[j % NBUF], dst.at[sv.at[j]], add=True)
    return body


def _sc_layer_body(x0, x1, ridx, cidx, einit, zer, binv_pad,
                   o2,
                   e_sh, o_sh, rv, cv, rows, esl, binv_v, gsem):
    c = lax.axis_index("c")
    s = lax.axis_index("s")

    pltpu.sync_copy(binv_pad.at[s], binv_v)

    sl = pl.ds(s * RPT, RPT)
    pltpu.sync_copy(einit.at[c, sl], e_sh.at[sl])
    pltpu.sync_copy(zer.at[sl], o_sh.at[sl])
    plsc.subcore_barrier()

    nblk = CH_W // IB
    base = s * CH_W

    @pl.when(c == 0)
    def _():
        pl.loop(0, nblk)(
            _ring_phase(x0, e_sh, ridx, cidx, rv, cv, rows, gsem, base))

    @pl.when(c == 1)
    def _():
        pl.loop(0, nblk)(
            _ring_phase(x1, e_sh, ridx, cidx, rv, cv, rows, gsem, base))

    plsc.subcore_barrier()

    @pl.loop(0, RPT // ESL)
    def _(p):
        psl = pl.ds(s * RPT + p * ESL, ESL)
        pltpu.sync_copy(e_sh.at[psl], esl)

        @pl.loop(0, ESL // 16)
        def _(t):
            vv = binv_v[pl.ds(p * ESL + t * 16, 16)]
            for k in range(16):
                sc = vv[k]
                r = t * 16 + k
                for j in range(H // 16):
                    esl[r, pl.ds(j * 16, 16)] = esl[r, pl.ds(j * 16, 16)] * sc

        vv = binv_v[pl.ds(p * ESL + (ESL // 16) * 16, 16)]
        for k in range(ESL - (ESL // 16) * 16):
            sc = vv[k]
            r = (ESL // 16) * 16 + k
            for j in range(H // 16):
                esl[r, pl.ds(j * 16, 16)] = esl[r, pl.ds(j * 16, 16)] * sc

        pltpu.sync_copy(esl, e_sh.at[psl])

    plsc.subcore_barrier()

    pl.loop(0, nblk)(
        _ring_phase(e_sh, o_sh, cidx, ridx, cv, rv, rows, gsem, base))

    plsc.subcore_barrier()
    pltpu.sync_copy(o_sh.at[sl], o2.at[c, sl])


def _sc_layer(x0, x1, ridx, cidx, einit, zer, binv_pad):
    return pl.kernel(
        _sc_layer_body,
        out_type=jax.ShapeDtypeStruct((NC, N, H), F32),
        mesh=_mesh,
        compiler_params=_sc_params,
        scratch_types=[
            pltpu.VMEM_SHARED((HE, H), F32),
            pltpu.VMEM_SHARED((N, H), F32),
            pltpu.VMEM((IB, K), jnp.int32),
            pltpu.VMEM((IB, K), jnp.int32),
            pltpu.VMEM((NBUF, K, H), F32),
            pltpu.VMEM((ESL, H), F32),
            pltpu.VMEM((640,), F32),
            pltpu.SemaphoreType.DMA((NBUF,)),
        ],
    )(x0, x1, ridx, cidx, einit, zer, binv_pad)


BN = 2000
NB = N // BN


def _tc_prep_body(xpad, fp, degB, degD, cntM, P16, batch, W1x, wfp, wm, Wp,
                  xts, EP, binv, dinv, cnt1, caa, cma):
    i = pl.program_id(0)
    cnt = cntM[0, :, 0:1] + cntM[1, :, 0:1]
    ism = jnp.minimum(cnt, 1.0)
    xt = (jnp.dot(xpad[...], W1x[...], precision=HI, preferred_element_type=F32)
          + fp[:, 0:1] * wfp[0:1, :] + fp[:, 1:2] * wfp[1:2, :] + ism * wm[...])
    xts[0] = xt[:, :H]
    xts[1] = xt[:, H:]
    Bdeg = degB[0, :, 0:1] + degB[1, :, 0:1]
    binv[...] = jnp.where(Bdeg > 0, 1.0 / Bdeg, 0.0)
    Ddeg = degD[0, :, 0:1] + degD[1, :, 0:1]
    dinv[...] = jnp.where(Ddeg > 0, 1.0 / Ddeg, 0.0)
    cnt1[...] = cnt
    P = P16[0, :, 0:4] + P16[1, :, 0:4]
    ep = jnp.dot(P, Wp[...], precision=HI, preferred_element_type=F32)
    EP[0] = ep[:, :H]
    EP[1] = ep[:, H:]
    oh = (lax.broadcasted_iota(jnp.int32, (G, BN), 0) == batch[0]).astype(F32)

    @pl.when(i == 0)
    def _():
        caa[...] = jnp.zeros((G, 1), F32)
        cma[...] = jnp.zeros((G, 1), F32)

    caa[...] += jnp.sum(oh, axis=1, keepdims=True)
    cma[...] += jnp.dot(oh, cnt, precision=HI, preferred_element_type=F32)

    @pl.when(i == NB - 1)
    def _():
        caa[...] = jnp.maximum(caa[...], 1.0)
        cma[...] = jnp.maximum(cma[...], 1.0)


def _tc_prep(xpad, fp, degB, degD, cntM, P16, batch1, W1x, wfp, wm, Wp):
    bs3 = pl.BlockSpec((NC, BN, 16), lambda i: (0, i, 0))
    return pl.pallas_call(
        _tc_prep_body,
        grid=(NB,),
        in_specs=[
            pl.BlockSpec((BN, 128), lambda i: (i, 0)),
            pl.BlockSpec((BN, 2), lambda i: (i, 0)),
            bs3, bs3, bs3, bs3,
            pl.BlockSpec((1, 1, BN), lambda i: (i, 0, 0)),
            pl.BlockSpec((128, NHID), lambda i: (0, 0)),
            pl.BlockSpec((2, NHID), lambda i: (0, 0)),
            pl.BlockSpec((1, NHID), lambda i: (0, 0)),
            pl.BlockSpec((4, NHID), lambda i: (0, 0)),
        ],
        out_specs=[
            pl.BlockSpec((NC, BN, H), lambda i: (0, i, 0)),
            pl.BlockSpec((NC, BN, H), lambda i: (0, i, 0)),
            pl.BlockSpec((BN, 1), lambda i: (i, 0)),
            pl.BlockSpec((BN, 1), lambda i: (i, 0)),
            pl.BlockSpec((BN, 1), lambda i: (i, 0)),
            pl.BlockSpec((G, 1), lambda i: (0, 0)),
            pl.BlockSpec((G, 1), lambda i: (0, 0)),
        ],
        out_shape=[
            jax.ShapeDtypeStruct((NC, N, H), F32),
            jax.ShapeDtypeStruct((NC, HE, H), F32),
            jax.ShapeDtypeStruct((N, 1), F32),
            jax.ShapeDtypeStruct((N, 1), F32),
            jax.ShapeDtypeStruct((N, 1), F32),
            jax.ShapeDtypeStruct((G, 1), F32),
            jax.ShapeDtypeStruct((G, 1), F32),
        ],
    )(xpad, fp, degB, degD, cntM, P16, batch1, W1x, wfp, wm, Wp)


def _pool_accum(i, oh, cnt, h0b, h1b, praw):
    @pl.when(i == 0)
    def _():
        praw[...] = jnp.zeros((4, G, H), F32)

    ch0 = cnt * h0b
    ch1 = cnt * h1b
    praw[0] += jnp.dot(oh, ch0, precision=HI, preferred_element_type=F32)
    praw[1] += jnp.dot(oh, ch1, precision=HI, preferred_element_type=F32)
    praw[2] += jnp.dot(oh, h0b, precision=HI, preferred_element_type=F32)
    praw[3] += jnp.dot(oh, h1b, precision=HI, preferred_element_type=F32)


def _tc_mid_body(o2, dinv, cnt1, batch, b, W, xts, praw):
    i = pl.program_id(0)
    dv = dinv[...]
    h0b = _leaky(o2[0] * dv + b[:, 0:H])
    h1b = _leaky(o2[1] * dv + b[:, H:])
    xts[0] = (jnp.dot(h0b, W[0:H, 0:H], precision=HI, preferred_element_type=F32)
              + jnp.dot(h1b, W[H:, 0:H], precision=HI, preferred_element_type=F32))
    xts[1] = (jnp.dot(h0b, W[0:H, H:], precision=HI, preferred_element_type=F32)
              + jnp.dot(h1b, W[H:, H:], precision=HI, preferred_element_type=F32))
    oh = (lax.broadcasted_iota(jnp.int32, (G, BN), 0) == batch[0]).astype(F32)
    _pool_accum(i, oh, cnt1[...], h0b, h1b, praw)


def _tc_last_body(o2, dinv, cnt1, batch, b, xts, praw):
    i = pl.program_id(0)
    dv = dinv[...]
    h0b = _leaky(o2[0] * dv + b[:, 0:H])
    h1b = _leaky(o2[1] * dv + b[:, H:])
    xts[0] = h0b
    xts[1] = h1b
    oh = (lax.broadcasted_iota(jnp.int32, (G, BN), 0) == batch[0]).astype(F32)
    _pool_accum(i, oh, cnt1[...], h0b, h1b, praw)


def _tc_layer_end(body, o2, dinv, cnt1, batch1, b, *Ws):
    in_specs = [
        pl.BlockSpec((NC, BN, H), lambda i: (0, i, 0)),
        pl.BlockSpec((BN, 1), lambda i: (i, 0)),
        pl.BlockSpec((BN, 1), lambda i: (i, 0)),
        pl.BlockSpec((1, 1, BN), lambda i: (i, 0, 0)),
        pl.BlockSpec((1, NHID), lambda i: (0, 0)),
    ] + [pl.BlockSpec((NHID, NHID), lambda i: (0, 0)) for _ in Ws]
    return pl.pallas_call(
        body,
        grid=(NB,),
        in_specs=in_specs,
        out_specs=[
            pl.BlockSpec((NC, BN, H), lambda i: (0, i, 0)),
            pl.BlockSpec((4, G, H), lambda i: (0, 0, 0)),
        ],
        out_shape=[
            jax.ShapeDtypeStruct((NC, N, H), F32),
            jax.ShapeDtypeStruct((4, G, H), F32),
        ],
    )(o2, dinv, cnt1, batch1, b, *Ws)


def _tc_final_body(p1, p2, p3, caa, cma, M1, mb1, M2, mb2, M3, mb3, out):
    t = jnp.zeros((G, NHID), F32)
    for q in range(4):
        zq = p1[q] + p2[q] + p3[q]
        zq = zq / (cma[...] if q < 2 else caa[...])
        t += jnp.dot(zq, M1[q * H:(q + 1) * H, :], precision=HI,
                     preferred_element_type=F32)
    t = _leaky(t + mb1[...])
    t = _leaky(jnp.dot(t, M2[...], precision=HI, preferred_element_type=F32)
               + mb2[...])
    out[...] = (jnp.dot(t, M3[...], precision=HI, preferred_element_type=F32)
                + mb3[...])


def _tc_final(p1, p2, p3, caa, cma, M1, mb1, M2, mb2, M3, mb3):
    return pl.pallas_call(
        _tc_final_body,
        out_shape=jax.ShapeDtypeStruct((G, NCLS), F32),
    )(p1, p2, p3, caa, cma, M1, mb1, M2, mb2, M3, mb3)


def kernel(x, fake_pos, pin_feature, edge_index, batch, macro_index,
           W1, Wp, b1, W2, b2, W3, b3, M1, mb1, M2, mb2, M3, mb3):
    row, col = edge_index[0], edge_index[1]

    xpad = jnp.pad(x, ((0, 0), (0, 3)))
    W1x = jnp.concatenate([W1[:125], jnp.zeros((3, NHID), F32)], axis=0)
    wfp = W1[125:127]
    wm = W1[127:128]
    batch1 = batch.reshape(NB, 1, BN)
    ridx = row.reshape(NCH, K)
    cidx = col.reshape(NCH, K)
    pin16 = jnp.pad(pin_feature, ((0, 0), (0, 12)))
    macro_pad = jnp.concatenate(
        [macro_index, jnp.full((12,), N, jnp.int32)]).reshape(NS, 32)
    zer = jnp.zeros((N, H), F32)
    zeros_e = jnp.zeros((NC, HE, H), F32)
    b1r, b2r, b3r = b1.reshape(1, NHID), b2.reshape(1, NHID), b3.reshape(1, NHID)
    mb1r, mb2r, mb3r = mb1.reshape(1, NHID), mb2.reshape(1, H), mb3.reshape(1, NCLS)

    degB, degD, cntM, P16 = _sc_precompute(ridx, cidx, pin16, macro_pad)
    xts1, EP, binv1, dinv1, cnt1, caa, cma = _tc_prep(
        xpad, fake_pos, degB, degD, cntM, P16, batch1, W1x, wfp, wm, Wp)
    binv_pad = jnp.pad(binv1.reshape(NS, RPT), ((0, 0), (0, 640 - RPT)))

    o2 = _sc_layer(xts1[0], xts1[1], ridx, cidx, EP, zer, binv_pad)
    xts2, praw1 = _tc_layer_end(_tc_mid_body, o2, dinv1, cnt1, batch1, b1r, W2)
    o2 = _sc_layer(xts2[0], xts2[1], ridx, cidx, zeros_e, zer, binv_pad)
    xts3, praw2 = _tc_layer_end(_tc_mid_body, o2, dinv1, cnt1, batch1, b2r, W3)
    o2 = _sc_layer(xts3[0], xts3[1], ridx, cidx, zeros_e, zer, binv_pad)
    _, praw3 = _tc_layer_end(_tc_last_body, o2, dinv1, cnt1, batch1, b3r)

    return _tc_final(praw1, praw2, praw3, caa, cma, M1, mb1r, M2, mb2r, M3, mb3r)


# speedup vs baseline: 6.5649x; 1.0628x over previous
import jax
import jax.numpy as jnp
from jax import lax
from jax.experimental import pallas as pl
from jax.experimental.pallas import tpu as pltpu
from jax.experimental.pallas import tpu_sc as plsc

N = 10000
E = 320000
HE = 10000
NHID = 128
H = 64
G = 16
NCLS = 4
SLOPE = 0.1

NC = 2
NS = 16
K = 80
NCH = E // K
CH_W = NCH // NS
CH_P = NCH // (NC * NS)
RPT = N // NS
NP16 = 10016
F32 = jnp.float32

_mesh = plsc.VectorSubcoreMesh(core_axis_name="c", subcore_axis_name="s")
_sc_params = pltpu.CompilerParams(use_tc_tiling_on_sc=False)


def _leaky(v):
    return jnp.where(v >= 0, v, SLOPE * v)


HI = jax.lax.Precision.HIGHEST


def _sc_pre_body(ridx, cidx, pin16, macro_pad,
                 degB_o, degD_o, cntM_o, P16_o,
                 accB, accD, accC, accP, rv, cv, pbuf, ones, zbuf, mbuf):
    c = lax.axis_index("c")
    s = lax.axis_index("s")
    w = c * NS + s

    pltpu.sync_copy(ridx.at[pl.ds(w * CH_P, CH_P)], rv)
    pltpu.sync_copy(cidx.at[pl.ds(w * CH_P, CH_P)], cv)

    @pl.loop(0, 626)
    def _(r):
        zbuf[r, pl.ds(0, 16)] = jnp.zeros((16,), F32)

    @pl.loop(0, K)
    def _(r):
        ones[r, pl.ds(0, 16)] = jnp.ones((16,), F32)

    for acc in (accB, accD, accC, accP):
        pltpu.sync_copy(zbuf, acc.at[pl.ds(s * 626, 626)])
    plsc.subcore_barrier()

    @pl.loop(0, CH_P)
    def _(ch):
        pltpu.sync_copy(pin16.at[pl.ds((w * CH_P + ch) * K, K)], pbuf)
        pltpu.sync_copy(ones, accD.at[rv.at[ch]], add=True)
        pltpu.sync_copy(ones, accB.at[cv.at[ch]], add=True)
        pltpu.sync_copy(pbuf, accP.at[cv.at[ch]], add=True)

    @pl.when(c == 0)
    def _():
        pltpu.sync_copy(macro_pad.at[s], mbuf)
        pltpu.sync_copy(ones.at[pl.ds(0, 32)], accC.at[mbuf], add=True)

    plsc.subcore_barrier()
    sl = pl.ds(s * RPT, RPT)
    pltpu.sync_copy(accB.at[sl], degB_o.at[c, sl])
    pltpu.sync_copy(accD.at[sl], degD_o.at[c, sl])
    pltpu.sync_copy(accC.at[sl], cntM_o.at[c, sl])
    pltpu.sync_copy(accP.at[sl], P16_o.at[c, sl])


def _sc_precompute(ridx, cidx, pin16, macro_pad):
    out = jax.ShapeDtypeStruct((NC, HE, 16), F32)
    return pl.kernel(
        _sc_pre_body,
        out_type=[out, out, out, out],
        mesh=_mesh,
        compiler_params=_sc_params,
        scratch_types=[
            pltpu.VMEM_SHARED((NP16, 16), F32),
            pltpu.VMEM_SHARED((NP16, 16), F32),
            pltpu.VMEM_SHARED((NP16, 16), F32),
            pltpu.VMEM_SHARED((NP16, 16), F32),
            pltpu.VMEM((CH_P, K), jnp.int32),
            pltpu.VMEM((CH_P, K), jnp.int32),
            pltpu.VMEM((K, 16), F32),
            pltpu.VMEM((K, 16), F32),
            pltpu.VMEM((626, 16), F32),
            pltpu.VMEM((32,), jnp.int32),
        ],
    )(ridx, cidx, pin16, macro_pad)


IB = 25
NBUF = 5
ESL = 125


def _ring_phase(src, dst, gidx_hbm, sidx_hbm, gv, sv, rows, gsem, ssem, base):
    def body(blk):
        pltpu.sync_copy(gidx_hbm.at[pl.ds(base + blk * IB, IB)], gv)
        pltpu.sync_copy(sidx_hbm.at[pl.ds(base + blk * IB, IB)], sv)
        gd = [None] * IB
        sd = [None] * IB
        for j in range(NBUF - 1):
            gd[j] = pltpu.async_copy(
                src.at[gv.at[j]], rows.at[j % NBUF], gsem.at[j % NBUF])
        for j in range(IB):
            q = j % NBUF
            gd[j].wait()
            sd[j] = pltpu.async_copy(rows.at[q], dst.at[sv.at[j]],
                                     ssem.at[q], add=True)
            nxt = j + NBUF - 1
            if nxt < IB:
                prev = nxt - NBUF
                if prev >= 0:
                    sd[prev].wait()
                gd[nxt] = pltpu.async_copy(
                    src.at[gv.at[nxt]], rows.at[nxt % NBUF],
                    gsem.at[nxt % NBUF])
        for j in range(max(0, IB - NBUF), IB):
            sd[j].wait()
    return body


def _sc_layer_body(x0, x1, ridx, cidx, einit, zer, binv_pad,
                   o2,
                   e_sh, o_sh, rv, cv, rows, esl, binv_v, gsem, ssem):
    c = lax.axis_index("c")
    s = lax.axis_index("s")

    pltpu.sync_copy(binv_pad.at[s], binv_v)

    sl = pl.ds(s * RPT, RPT)
    pltpu.sync_copy(einit.at[c, sl], e_sh.at[sl])
    pltpu.sync_copy(zer.at[sl], o_sh.at[sl])
    plsc.subcore_barrier()

    nblk = CH_W // IB
    base = s * CH_W

    @pl.when(c == 0)
    def _():
        pl.loop(0, nblk)(
            _ring_phase(x0, e_sh, ridx, cidx, rv, cv, rows, gsem, ssem, base))

    @pl.when(c == 1)
    def _():
        pl.loop(0, nblk)(
            _ring_phase(x1, e_sh, ridx, cidx, rv, cv, rows, gsem, ssem, base))

    plsc.subcore_barrier()

    @pl.loop(0, RPT // ESL)
    def _(p):
        psl = pl.ds(s * RPT + p * ESL, ESL)
        pltpu.sync_copy(e_sh.at[psl], esl)

        @pl.loop(0, ESL // 16)
        def _(t):
            vv = binv_v[pl.ds(p * ESL + t * 16, 16)]
            for k in range(16):
                sc = vv[k]
                r = t * 16 + k
                for j in range(H // 16):
                    esl[r, pl.ds(j * 16, 16)] = esl[r, pl.ds(j * 16, 16)] * sc

        vv = binv_v[pl.ds(p * ESL + (ESL // 16) * 16, 16)]
        for k in range(ESL - (ESL // 16) * 16):
            sc = vv[k]
            r = (ESL // 16) * 16 + k
            for j in range(H // 16):
                esl[r, pl.ds(j * 16, 16)] = esl[r, pl.ds(j * 16, 16)] * sc

        pltpu.sync_copy(esl, e_sh.at[psl])

    plsc.subcore_barrier()

    pl.loop(0, nblk)(
        _ring_phase(e_sh, o_sh, cidx, ridx, cv, rv, rows, gsem, ssem, base))

    plsc.subcore_barrier()
    pltpu.sync_copy(o_sh.at[sl], o2.at[c, sl])


def _sc_layer(x0, x1, ridx, cidx, einit, zer, binv_pad):
    return pl.kernel(
        _sc_layer_body,
        out_type=jax.ShapeDtypeStruct((NC, N, H), F32),
        mesh=_mesh,
        compiler_params=_sc_params,
        scratch_types=[
            pltpu.VMEM_SHARED((HE, H), F32),
            pltpu.VMEM_SHARED((N, H), F32),
            pltpu.VMEM((IB, K), jnp.int32),
            pltpu.VMEM((IB, K), jnp.int32),
            pltpu.VMEM((NBUF, K, H), F32),
            pltpu.VMEM((ESL, H), F32),
            pltpu.VMEM((640,), F32),
            pltpu.SemaphoreType.DMA((NBUF,)),
            pltpu.SemaphoreType.DMA((NBUF,)),
        ],
    )(x0, x1, ridx, cidx, einit, zer, binv_pad)


BN = 2000
NB = N // BN


def _tc_prep_body(xpad, fp, degB, degD, cntM, P16, batch, W1x, wfp, wm, Wp,
                  xts, EP, binv, dinv, cnt1, caa, cma):
    i = pl.program_id(0)
    cnt = cntM[0, :, 0:1] + cntM[1, :, 0:1]
    ism = jnp.minimum(cnt, 1.0)
    xt = (jnp.dot(xpad[...], W1x[...], precision=HI, preferred_element_type=F32)
          + fp[:, 0:1] * wfp[0:1, :] + fp[:, 1:2] * wfp[1:2, :] + ism * wm[...])
    xts[0] = xt[:, :H]
    xts[1] = xt[:, H:]
    Bdeg = degB[0, :, 0:1] + degB[1, :, 0:1]
    binv[...] = jnp.where(Bdeg > 0, 1.0 / Bdeg, 0.0)
    Ddeg = degD[0, :, 0:1] + degD[1, :, 0:1]
    dinv[...] = jnp.where(Ddeg > 0, 1.0 / Ddeg, 0.0)
    cnt1[...] = cnt
    P = P16[0, :, 0:4] + P16[1, :, 0:4]
    ep = jnp.dot(P, Wp[...], precision=HI, preferred_element_type=F32)
    EP[0] = ep[:, :H]
    EP[1] = ep[:, H:]
    oh = (lax.broadcasted_iota(jnp.int32, (G, BN), 0) == batch[0]).astype(F32)

    @pl.when(i == 0)
    def _():
        caa[...] = jnp.zeros((G, 1), F32)
        cma[...] = jnp.zeros((G, 1), F32)

    caa[...] += jnp.sum(oh, axis=1, keepdims=True)
    cma[...] += jnp.dot(oh, cnt, precision=HI, preferred_element_type=F32)

    @pl.when(i == NB - 1)
    def _():
        caa[...] = jnp.maximum(caa[...], 1.0)
        cma[...] = jnp.maximum(cma[...], 1.0)


def _tc_prep(xpad, fp, degB, degD, cntM, P16, batch1, W1x, wfp, wm, Wp):
    bs3 = pl.BlockSpec((NC, BN, 16), lambda i: (0, i, 0))
    return pl.pallas_call(
        _tc_prep_body,
        grid=(NB,),
        in_specs=[
            pl.BlockSpec((BN, 128), lambda i: (i, 0)),
            pl.BlockSpec((BN, 2), lambda i: (i, 0)),
            bs3, bs3, bs3, bs3,
            pl.BlockSpec((1, 1, BN), lambda i: (i, 0, 0)),
            pl.BlockSpec((128, NHID), lambda i: (0, 0)),
            pl.BlockSpec((2, NHID), lambda i: (0, 0)),
            pl.BlockSpec((1, NHID), lambda i: (0, 0)),
            pl.BlockSpec((4, NHID), lambda i: (0, 0)),
        ],
        out_specs=[
            pl.BlockSpec((NC, BN, H), lambda i: (0, i, 0)),
            pl.BlockSpec((NC, BN, H), lambda i: (0, i, 0)),
            pl.BlockSpec((BN, 1), lambda i: (i, 0)),
            pl.BlockSpec((BN, 1), lambda i: (i, 0)),
            pl.BlockSpec((BN, 1), lambda i: (i, 0)),
            pl.BlockSpec((G, 1), lambda i: (0, 0)),
            pl.BlockSpec((G, 1), lambda i: (0, 0)),
        ],
        out_shape=[
            jax.ShapeDtypeStruct((NC, N, H), F32),
            jax.ShapeDtypeStruct((NC, HE, H), F32),
            jax.ShapeDtypeStruct((N, 1), F32),
            jax.ShapeDtypeStruct((N, 1), F32),
            jax.ShapeDtypeStruct((N, 1), F32),
            jax.ShapeDtypeStruct((G, 1), F32),
            jax.ShapeDtypeStruct((G, 1), F32),
        ],
    )(xpad, fp, degB, degD, cntM, P16, batch1, W1x, wfp, wm, Wp)


def _pool_accum(i, oh, cnt, h0b, h1b, praw):
    @pl.when(i == 0)
    def _():
        praw[...] = jnp.zeros((4, G, H), F32)

    ch0 = cnt * h0b
    ch1 = cnt * h1b
    praw[0] += jnp.dot(oh, ch0, precision=HI, preferred_element_type=F32)
    praw[1] += jnp.dot(oh, ch1, precision=HI, preferred_element_type=F32)
    praw[2] += jnp.dot(oh, h0b, precision=HI, preferred_element_type=F32)
    praw[3] += jnp.dot(oh, h1b, precision=HI, preferred_element_type=F32)


def _tc_mid_body(o2, dinv, cnt1, batch, b, W, xts, praw):
    i = pl.program_id(0)
    dv = dinv[...]
    h0b = _leaky(o2[0] * dv + b[:, 0:H])
    h1b = _leaky(o2[1] * dv + b[:, H:])
    xts[0] = (jnp.dot(h0b, W[0:H, 0:H], precision=HI, preferred_element_type=F32)
              + jnp.dot(h1b, W[H:, 0:H], precision=HI, preferred_element_type=F32))
    xts[1] = (jnp.dot(h0b, W[0:H, H:], precision=HI, preferred_element_type=F32)
              + jnp.dot(h1b, W[H:, H:], precision=HI, preferred_element_type=F32))
    oh = (lax.broadcasted_iota(jnp.int32, (G, BN), 0) == batch[0]).astype(F32)
    _pool_accum(i, oh, cnt1[...], h0b, h1b, praw)


def _tc_last_body(o2, dinv, cnt1, batch, b, xts, praw):
    i = pl.program_id(0)
    dv = dinv[...]
    h0b = _leaky(o2[0] * dv + b[:, 0:H])
    h1b = _leaky(o2[1] * dv + b[:, H:])
    xts[0] = h0b
    xts[1] = h1b
    oh = (lax.broadcasted_iota(jnp.int32, (G, BN), 0) == batch[0]).astype(F32)
    _pool_accum(i, oh, cnt1[...], h0b, h1b, praw)


def _tc_layer_end(body, o2, dinv, cnt1, batch1, b, *Ws):
    in_specs = [
        pl.BlockSpec((NC, BN, H), lambda i: (0, i, 0)),
        pl.BlockSpec((BN, 1), lambda i: (i, 0)),
        pl.BlockSpec((BN, 1), lambda i: (i, 0)),
        pl.BlockSpec((1, 1, BN), lambda i: (i, 0, 0)),
        pl.BlockSpec((1, NHID), lambda i: (0, 0)),
    ] + [pl.BlockSpec((NHID, NHID), lambda i: (0, 0)) for _ in Ws]
    return pl.pallas_call(
        body,
        grid=(NB,),
        in_specs=in_specs,
        out_specs=[
            pl.BlockSpec((NC, BN, H), lambda i: (0, i, 0)),
            pl.BlockSpec((4, G, H), lambda i: (0, 0, 0)),
        ],
        out_shape=[
            jax.ShapeDtypeStruct((NC, N, H), F32),
            jax.ShapeDtypeStruct((4, G, H), F32),
        ],
    )(o2, dinv, cnt1, batch1, b, *Ws)


def _tc_final_body(p1, p2, p3, caa, cma, M1, mb1, M2, mb2, M3, mb3, out):
    t = jnp.zeros((G, NHID), F32)
    for q in range(4):
        zq = p1[q] + p2[q] + p3[q]
        zq = zq / (cma[...] if q < 2 else caa[...])
        t += jnp.dot(zq, M1[q * H:(q + 1) * H, :], precision=HI,
                     preferred_element_type=F32)
    t = _leaky(t + mb1[...])
    t = _leaky(jnp.dot(t, M2[...], precision=HI, preferred_element_type=F32)
               + mb2[...])
    out[...] = (jnp.dot(t, M3[...], precision=HI, preferred_element_type=F32)
                + mb3[...])


def _tc_final(p1, p2, p3, caa, cma, M1, mb1, M2, mb2, M3, mb3):
    return pl.pallas_call(
        _tc_final_body,
        out_shape=jax.ShapeDtypeStruct((G, NCLS), F32),
    )(p1, p2, p3, caa, cma, M1, mb1, M2, mb2, M3, mb3)


def kernel(x, fake_pos, pin_feature, edge_index, batch, macro_index,
           W1, Wp, b1, W2, b2, W3, b3, M1, mb1, M2, mb2, M3, mb3):
    row, col = edge_index[0], edge_index[1]

    xpad = jnp.pad(x, ((0, 0), (0, 3)))
    W1x = jnp.concatenate([W1[:125], jnp.zeros((3, NHID), F32)], axis=0)
    wfp = W1[125:127]
    wm = W1[127:128]
    batch1 = batch.reshape(NB, 1, BN)
    ridx = row.reshape(NCH, K)
    cidx = col.reshape(NCH, K)
    pin16 = jnp.pad(pin_feature, ((0, 0), (0, 12)))
    macro_pad = jnp.concatenate(
        [macro_index, jnp.full((12,), N, jnp.int32)]).reshape(NS, 32)
    zer = jnp.zeros((N, H), F32)
    zeros_e = jnp.zeros((NC, HE, H), F32)
    b1r, b2r, b3r = b1.reshape(1, NHID), b2.reshape(1, NHID), b3.reshape(1, NHID)
    mb1r, mb2r, mb3r = mb1.reshape(1, NHID), mb2.reshape(1, H), mb3.reshape(1, NCLS)

    degB, degD, cntM, P16 = _sc_precompute(ridx, cidx, pin16, macro_pad)
    xts1, EP, binv1, dinv1, cnt1, caa, cma = _tc_prep(
        xpad, fake_pos, degB, degD, cntM, P16, batch1, W1x, wfp, wm, Wp)
    binv_pad = jnp.pad(binv1.reshape(NS, RPT), ((0, 0), (0, 640 - RPT)))

    o2 = _sc_layer(xts1[0], xts1[1], ridx, cidx, EP, zer, binv_pad)
    xts2, praw1 = _tc_layer_end(_tc_mid_body, o2, dinv1, cnt1, batch1, b1r, W2)
    o2 = _sc_layer(xts2[0], xts2[1], ridx, cidx, zeros_e, zer, binv_pad)
    xts3, praw2 = _tc_layer_end(_tc_mid_body, o2, dinv1, cnt1, batch1, b2r, W3)
    o2 = _sc_layer(xts3[0], xts3[1], ridx, cidx, zeros_e, zer, binv_pad)
    _, praw3 = _tc_layer_end(_tc_last_body, o2, dinv1, cnt1, batch1, b3r)

    return _tc_final(praw1, praw2, praw3, caa, cma, M1, mb1r, M2, mb2r, M3, mb3r)


# speedup vs baseline: 7.1290x; 1.0859x over previous
import jax
import jax.numpy as jnp
from jax import lax
from jax.experimental import pallas as pl
from jax.experimental.pallas import tpu as pltpu
from jax.experimental.pallas import tpu_sc as plsc

N = 10000
E = 320000
HE = 10000
NHID = 128
H = 64
G = 16
NCLS = 4
SLOPE = 0.1

NC = 2
NS = 16
K = 128
NCH = E // K
CH_W = NCH // NS
CH_P = NCH // (NC * NS)
RPT = N // NS
NP16 = 10016
F32 = jnp.float32

_mesh = plsc.VectorSubcoreMesh(core_axis_name="c", subcore_axis_name="s")
_sc_params = pltpu.CompilerParams(use_tc_tiling_on_sc=False)


def _leaky(v):
    return jnp.where(v >= 0, v, SLOPE * v)


HI = jax.lax.Precision.DEFAULT


def _sc_pre_body(ridx, cidx, pin16, macro_pad,
                 degB_o, degD_o, cntM_o, P16_o,
                 accB, accD, accC, accP, rv, cv, pbuf, ones, zbuf, mbuf):
    c = lax.axis_index("c")
    s = lax.axis_index("s")
    w = c * NS + s
    base = w * CH_P + jnp.minimum(w, 4)
    nch = CH_P + jnp.where(w < 4, 1, 0)

    pltpu.sync_copy(ridx.at[pl.ds(base, CH_P)], rv.at[pl.ds(0, CH_P)])
    pltpu.sync_copy(cidx.at[pl.ds(base, CH_P)], cv.at[pl.ds(0, CH_P)])

    @pl.when(w < 4)
    def _():
        pltpu.sync_copy(ridx.at[pl.ds(base + CH_P, 1)], rv.at[pl.ds(CH_P, 1)])
        pltpu.sync_copy(cidx.at[pl.ds(base + CH_P, 1)], cv.at[pl.ds(CH_P, 1)])

    @pl.loop(0, 626)
    def _(r):
        zbuf[r, pl.ds(0, 16)] = jnp.zeros((16,), F32)

    @pl.loop(0, K)
    def _(r):
        ones[r, pl.ds(0, 16)] = jnp.ones((16,), F32)

    for acc in (accB, accD, accC, accP):
        pltpu.sync_copy(zbuf, acc.at[pl.ds(s * 626, 626)])
    plsc.subcore_barrier()

    @pl.loop(0, nch)
    def _(ch):
        pltpu.sync_copy(pin16.at[pl.ds((base + ch) * K, K)], pbuf)
        pltpu.sync_copy(ones, accD.at[rv.at[ch]], add=True)
        pltpu.sync_copy(ones, accB.at[cv.at[ch]], add=True)
        pltpu.sync_copy(pbuf, accP.at[cv.at[ch]], add=True)

    @pl.when(c == 0)
    def _():
        pltpu.sync_copy(macro_pad.at[s], mbuf)
        pltpu.sync_copy(ones.at[pl.ds(0, 32)], accC.at[mbuf], add=True)

    plsc.subcore_barrier()
    sl = pl.ds(s * RPT, RPT)
    pltpu.sync_copy(accB.at[sl], degB_o.at[c, sl])
    pltpu.sync_copy(accD.at[sl], degD_o.at[c, sl])
    pltpu.sync_copy(accC.at[sl], cntM_o.at[c, sl])
    pltpu.sync_copy(accP.at[sl], P16_o.at[c, sl])


def _sc_precompute(ridx, cidx, pin16, macro_pad):
    out = jax.ShapeDtypeStruct((NC, HE, 16), F32)
    return pl.kernel(
        _sc_pre_body,
        out_type=[out, out, out, out],
        mesh=_mesh,
        compiler_params=_sc_params,
        scratch_types=[
            pltpu.VMEM_SHARED((NP16, 16), F32),
            pltpu.VMEM_SHARED((NP16, 16), F32),
            pltpu.VMEM_SHARED((NP16, 16), F32),
            pltpu.VMEM_SHARED((NP16, 16), F32),
            pltpu.VMEM((CH_P + 1, K), jnp.int32),
            pltpu.VMEM((CH_P + 1, K), jnp.int32),
            pltpu.VMEM((K, 16), F32),
            pltpu.VMEM((K, 16), F32),
            pltpu.VMEM((626, 16), F32),
            pltpu.VMEM((32,), jnp.int32),
        ],
    )(ridx, cidx, pin16, macro_pad)


IB = 26
NBUF = 4
ESL = 125


def _ring_phase(src, dst, gidx_hbm, sidx_hbm, gv, sv, rows, gsem, ssem, base):
    def body(blk):
        pltpu.sync_copy(gidx_hbm.at[pl.ds(base + blk * IB, IB)], gv)
        pltpu.sync_copy(sidx_hbm.at[pl.ds(base + blk * IB, IB)], sv)
        gd = [None] * IB
        sd = [None] * IB
        for j in range(NBUF - 1):
            gd[j] = pltpu.async_copy(
                src.at[gv.at[j]], rows.at[j % NBUF], gsem.at[j % NBUF])
        for j in range(IB):
            q = j % NBUF
            gd[j].wait()
            sd[j] = pltpu.async_copy(rows.at[q], dst.at[sv.at[j]],
                                     ssem.at[q], add=True)
            nxt = j + NBUF - 1
            if nxt < IB:
                prev = nxt - NBUF
                if prev >= 0:
                    sd[prev].wait()
                gd[nxt] = pltpu.async_copy(
                    src.at[gv.at[nxt]], rows.at[nxt % NBUF],
                    gsem.at[nxt % NBUF])
        for j in range(max(0, IB - NBUF), IB):
            sd[j].wait()
    return body


def _tail_chunk(src, dst, gidx_hbm, sidx_hbm, gv, sv, rows, row):
    pltpu.sync_copy(gidx_hbm.at[pl.ds(row, 1)], gv.at[pl.ds(0, 1)])
    pltpu.sync_copy(sidx_hbm.at[pl.ds(row, 1)], sv.at[pl.ds(0, 1)])
    pltpu.sync_copy(src.at[gv.at[0]], rows.at[0])
    pltpu.sync_copy(rows.at[0], dst.at[sv.at[0]], add=True)


def _sc_layer_body(x0, x1, ridx, cidx, einit, zer, binv_pad,
                   o2,
                   e_sh, o_sh, rv, cv, rows, esl, binv_v, gsem, ssem):
    c = lax.axis_index("c")
    s = lax.axis_index("s")

    pltpu.sync_copy(binv_pad.at[s], binv_v)

    sl = pl.ds(s * RPT, RPT)
    pltpu.sync_copy(einit.at[c, sl], e_sh.at[sl])
    pltpu.sync_copy(zer.at[sl], o_sh.at[sl])
    plsc.subcore_barrier()

    nblk = CH_W // IB
    base = s * CH_W + jnp.minimum(s, 4)
    tail = base + CH_W

    @pl.when(c == 0)
    def _():
        pl.loop(0, nblk)(
            _ring_phase(x0, e_sh, ridx, cidx, rv, cv, rows, gsem, ssem, base))

        @pl.when(s < 4)
        def _():
            _tail_chunk(x0, e_sh, ridx, cidx, rv, cv, rows, tail)

    @pl.when(c == 1)
    def _():
        pl.loop(0, nblk)(
            _ring_phase(x1, e_sh, ridx, cidx, rv, cv, rows, gsem, ssem, base))

        @pl.when(s < 4)
        def _():
            _tail_chunk(x1, e_sh, ridx, cidx, rv, cv, rows, tail)

    plsc.subcore_barrier()

    @pl.loop(0, RPT // ESL)
    def _(p):
        psl = pl.ds(s * RPT + p * ESL, ESL)
        pltpu.sync_copy(e_sh.at[psl], esl)

        @pl.loop(0, ESL // 16)
        def _(t):
            vv = binv_v[pl.ds(p * ESL + t * 16, 16)]
            for k in range(16):
                sc = vv[k]
                r = t * 16 + k
                for j in range(H // 16):
                    esl[r, pl.ds(j * 16, 16)] = esl[r, pl.ds(j * 16, 16)] * sc

        vv = binv_v[pl.ds(p * ESL + (ESL // 16) * 16, 16)]
        for k in range(ESL - (ESL // 16) * 16):
            sc = vv[k]
            r = (ESL // 16) * 16 + k
            for j in range(H // 16):
                esl[r, pl.ds(j * 16, 16)] = esl[r, pl.ds(j * 16, 16)] * sc

        pltpu.sync_copy(esl, e_sh.at[psl])

    plsc.subcore_barrier()

    pl.loop(0, nblk)(
        _ring_phase(e_sh, o_sh, cidx, ridx, cv, rv, rows, gsem, ssem, base))

    @pl.when(s < 4)
    def _():
        _tail_chunk(e_sh, o_sh, cidx, ridx, cv, rv, rows, tail)

    plsc.subcore_barrier()
    pltpu.sync_copy(o_sh.at[sl], o2.at[c, sl])


def _sc_layer(x0, x1, ridx, cidx, einit, zer, binv_pad):
    return pl.kernel(
        _sc_layer_body,
        out_type=jax.ShapeDtypeStruct((NC, N, H), F32),
        mesh=_mesh,
        compiler_params=_sc_params,
        scratch_types=[
            pltpu.VMEM_SHARED((HE, H), F32),
            pltpu.VMEM_SHARED((N, H), F32),
            pltpu.VMEM((IB, K), jnp.int32),
            pltpu.VMEM((IB, K), jnp.int32),
            pltpu.VMEM((NBUF, K, H), F32),
            pltpu.VMEM((ESL, H), F32),
            pltpu.VMEM((640,), F32),
            pltpu.SemaphoreType.DMA((NBUF,)),
            pltpu.SemaphoreType.DMA((NBUF,)),
        ],
    )(x0, x1, ridx, cidx, einit, zer, binv_pad)


BN = 2000
NB = N // BN


def _tc_prep_body(xpad, fp, degB, degD, cntM, P16, batch, W1x, wfp, wm, Wp,
                  xt0, xt1, EP, binv, dinv, cnt1, caa, cma):
    i = pl.program_id(0)
    cnt = cntM[0, :, 0:1] + cntM[1, :, 0:1]
    ism = jnp.minimum(cnt, 1.0)
    xt = (jnp.dot(xpad[...], W1x[...], precision=HI, preferred_element_type=F32)
          + fp[:, 0:1] * wfp[0:1, :] + fp[:, 1:2] * wfp[1:2, :] + ism * wm[...])
    xt0[...] = xt[:, :H]
    xt1[...] = xt[:, H:]
    Bdeg = degB[0, :, 0:1] + degB[1, :, 0:1]
    binv[...] = jnp.where(Bdeg > 0, 1.0 / Bdeg, 0.0)
    Ddeg = degD[0, :, 0:1] + degD[1, :, 0:1]
    dinv[...] = jnp.where(Ddeg > 0, 1.0 / Ddeg, 0.0)
    cnt1[...] = cnt
    P = P16[0, :, 0:4] + P16[1, :, 0:4]
    ep = jnp.dot(P, Wp[...], precision=HI, preferred_element_type=F32)
    EP[0] = ep[:, :H]
    EP[1] = ep[:, H:]
    oh = (lax.broadcasted_iota(jnp.int32, (G, BN), 0) == batch[0]).astype(F32)

    @pl.when(i == 0)
    def _():
        caa[...] = jnp.zeros((G, 1), F32)
        cma[...] = jnp.zeros((G, 1), F32)

    caa[...] += jnp.sum(oh, axis=1, keepdims=True)
    cma[...] += jnp.dot(oh, cnt, precision=HI, preferred_element_type=F32)

    @pl.when(i == NB - 1)
    def _():
        caa[...] = jnp.maximum(caa[...], 1.0)
        cma[...] = jnp.maximum(cma[...], 1.0)


def _tc_prep(xpad, fp, degB, degD, cntM, P16, batch1, W1x, wfp, wm, Wp):
    bs3 = pl.BlockSpec((NC, BN, 16), lambda i: (0, i, 0))
    return pl.pallas_call(
        _tc_prep_body,
        grid=(NB,),
        in_specs=[
            pl.BlockSpec((BN, 128), lambda i: (i, 0)),
            pl.BlockSpec((BN, 2), lambda i: (i, 0)),
            bs3, bs3, bs3, bs3,
            pl.BlockSpec((1, 1, BN), lambda i: (i, 0, 0)),
            pl.BlockSpec((128, NHID), lambda i: (0, 0)),
            pl.BlockSpec((2, NHID), lambda i: (0, 0)),
            pl.BlockSpec((1, NHID), lambda i: (0, 0)),
            pl.BlockSpec((4, NHID), lambda i: (0, 0)),
        ],
        out_specs=[
            pl.BlockSpec((BN, H), lambda i: (i, 0)),
            pl.BlockSpec((BN, H), lambda i: (i, 0)),
            pl.BlockSpec((NC, BN, H), lambda i: (0, i, 0)),
            pl.BlockSpec((BN, 1), lambda i: (i, 0)),
            pl.BlockSpec((BN, 1), lambda i: (i, 0)),
            pl.BlockSpec((BN, 1), lambda i: (i, 0)),
            pl.BlockSpec((G, 1), lambda i: (0, 0)),
            pl.BlockSpec((G, 1), lambda i: (0, 0)),
        ],
        out_shape=[
            jax.ShapeDtypeStruct((N, H), F32),
            jax.ShapeDtypeStruct((N, H), F32),
            jax.ShapeDtypeStruct((NC, HE, H), F32),
            jax.ShapeDtypeStruct((N, 1), F32),
            jax.ShapeDtypeStruct((N, 1), F32),
            jax.ShapeDtypeStruct((N, 1), F32),
            jax.ShapeDtypeStruct((G, 1), F32),
            jax.ShapeDtypeStruct((G, 1), F32),
        ],
    )(xpad, fp, degB, degD, cntM, P16, batch1, W1x, wfp, wm, Wp)


def _pool_accum(i, oh, cnt, h0b, h1b, praw):
    @pl.when(i == 0)
    def _():
        praw[...] = jnp.zeros((4, G, H), F32)

    ch0 = cnt * h0b
    ch1 = cnt * h1b
    praw[0] += jnp.dot(oh, ch0, precision=HI, preferred_element_type=F32)
    praw[1] += jnp.dot(oh, ch1, precision=HI, preferred_element_type=F32)
    praw[2] += jnp.dot(oh, h0b, precision=HI, preferred_element_type=F32)
    praw[3] += jnp.dot(oh, h1b, precision=HI, preferred_element_type=F32)


def _tc_mid_body(o2, dinv, cnt1, batch, b, W, xt0, xt1, praw):
    i = pl.program_id(0)
    dv = dinv[...]
    h0b = _leaky(o2[0] * dv + b[:, 0:H])
    h1b = _leaky(o2[1] * dv + b[:, H:])
    xt0[...] = (jnp.dot(h0b, W[0:H, 0:H], precision=HI, preferred_element_type=F32)
                + jnp.dot(h1b, W[H:, 0:H], precision=HI, preferred_element_type=F32))
    xt1[...] = (jnp.dot(h0b, W[0:H, H:], precision=HI, preferred_element_type=F32)
                + jnp.dot(h1b, W[H:, H:], precision=HI, preferred_element_type=F32))
    oh = (lax.broadcasted_iota(jnp.int32, (G, BN), 0) == batch[0]).astype(F32)
    _pool_accum(i, oh, cnt1[...], h0b, h1b, praw)


def _tc_last_body(o2, dinv, cnt1, batch, b, xt0, xt1, praw):
    i = pl.program_id(0)
    dv = dinv[...]
    h0b = _leaky(o2[0] * dv + b[:, 0:H])
    h1b = _leaky(o2[1] * dv + b[:, H:])
    xt0[...] = h0b
    xt1[...] = h1b
    oh = (lax.broadcasted_iota(jnp.int32, (G, BN), 0) == batch[0]).astype(F32)
    _pool_accum(i, oh, cnt1[...], h0b, h1b, praw)


def _tc_layer_end(body, o2, dinv, cnt1, batch1, b, *Ws):
    in_specs = [
        pl.BlockSpec((NC, BN, H), lambda i: (0, i, 0)),
        pl.BlockSpec((BN, 1), lambda i: (i, 0)),
        pl.BlockSpec((BN, 1), lambda i: (i, 0)),
        pl.BlockSpec((1, 1, BN), lambda i: (i, 0, 0)),
        pl.BlockSpec((1, NHID), lambda i: (0, 0)),
    ] + [pl.BlockSpec((NHID, NHID), lambda i: (0, 0)) for _ in Ws]
    return pl.pallas_call(
        body,
        grid=(NB,),
        in_specs=in_specs,
        out_specs=[
            pl.BlockSpec((BN, H), lambda i: (i, 0)),
            pl.BlockSpec((BN, H), lambda i: (i, 0)),
            pl.BlockSpec((4, G, H), lambda i: (0, 0, 0)),
        ],
        out_shape=[
            jax.ShapeDtypeStruct((N, H), F32),
            jax.ShapeDtypeStruct((N, H), F32),
            jax.ShapeDtypeStruct((4, G, H), F32),
        ],
    )(o2, dinv, cnt1, batch1, b, *Ws)


def _tc_final_body(p1, p2, p3, caa, cma, M1, mb1, M2, mb2, M3, mb3, out):
    t = jnp.zeros((G, NHID), F32)
    for q in range(4):
        zq = p1[q] + p2[q] + p3[q]
        zq = zq / (cma[...] if q < 2 else caa[...])
        t += jnp.dot(zq, M1[q * H:(q + 1) * H, :], precision=HI,
                     preferred_element_type=F32)
    t = _leaky(t + mb1[...])
    t = _leaky(jnp.dot(t, M2[...], precision=HI, preferred_element_type=F32)
               + mb2[...])
    out[...] = (jnp.dot(t, M3[...], precision=HI, preferred_element_type=F32)
                + mb3[...])


def _tc_pin_body(pin, out):
    out[...] = jnp.concatenate(
        [pin[...], jnp.zeros((BE, 12), F32)], axis=1)


BE = 8000


def _tc_pin(pin_feature):
    return pl.pallas_call(
        _tc_pin_body,
        grid=(E // BE,),
        in_specs=[pl.BlockSpec((BE, 4), lambda i: (i, 0))],
        out_specs=pl.BlockSpec((BE, 16), lambda i: (i, 0)),
        out_shape=jax.ShapeDtypeStruct((E, 16), F32),
    )(pin_feature)


def _tc_final(p1, p2, p3, caa, cma, M1, mb1, M2, mb2, M3, mb3):
    return pl.pallas_call(
        _tc_final_body,
        out_shape=jax.ShapeDtypeStruct((G, NCLS), F32),
    )(p1, p2, p3, caa, cma, M1, mb1, M2, mb2, M3, mb3)


def kernel(x, fake_pos, pin_feature, edge_index, batch, macro_index,
           W1, Wp, b1, W2, b2, W3, b3, M1, mb1, M2, mb2, M3, mb3):
    row, col = edge_index[0], edge_index[1]

    xpad = jnp.pad(x, ((0, 0), (0, 3)))
    W1x = jnp.concatenate([W1[:125], jnp.zeros((3, NHID), F32)], axis=0)
    wfp = W1[125:127]
    wm = W1[127:128]
    batch1 = batch.reshape(NB, 1, BN)
    ridx = row.reshape(NCH, K)
    cidx = col.reshape(NCH, K)
    pin16 = _tc_pin(pin_feature)
    macro_pad = jnp.concatenate(
        [macro_index, jnp.full((12,), N, jnp.int32)]).reshape(NS, 32)
    zer = jnp.zeros((N, H), F32)
    zeros_e = jnp.zeros((NC, HE, H), F32)
    b1r, b2r, b3r = b1.reshape(1, NHID), b2.reshape(1, NHID), b3.reshape(1, NHID)
    mb1r, mb2r, mb3r = mb1.reshape(1, NHID), mb2.reshape(1, H), mb3.reshape(1, NCLS)

    degB, degD, cntM, P16 = _sc_precompute(ridx, cidx, pin16, macro_pad)
    x0, x1, EP, binv1, dinv1, cnt1, caa, cma = _tc_prep(
        xpad, fake_pos, degB, degD, cntM, P16, batch1, W1x, wfp, wm, Wp)
    binv_pad = jnp.pad(binv1.reshape(NS, RPT), ((0, 0), (0, 640 - RPT)))

    o2 = _sc_layer(x0, x1, ridx, cidx, EP, zer, binv_pad)
    x0, x1, praw1 = _tc_layer_end(_tc_mid_body, o2, dinv1, cnt1, batch1, b1r, W2)
    o2 = _sc_layer(x0, x1, ridx, cidx, zeros_e, zer, binv_pad)
    x0, x1, praw2 = _tc_layer_end(_tc_mid_body, o2, dinv1, cnt1, batch1, b2r, W3)
    o2 = _sc_layer(x0, x1, ridx, cidx, zeros_e, zer, binv_pad)
    _, _, praw3 = _tc_layer_end(_tc_last_body, o2, dinv1, cnt1, batch1, b3r)

    return _tc_final(praw1, praw2, praw3, caa, cma, M1, mb1r, M2, mb2r, M3, mb3r)
